# Initial kernel scaffold; baseline (speedup 1.0000x reference)
import jax
import jax.numpy as jnp
from jax.experimental import pallas as pl


def kernel(x, edge_index, W, b):
    raise NotImplementedError("write your pallas kernel here")


# speedup vs baseline: 10.8816x; 10.8816x over previous
import functools

import jax
import jax.numpy as jnp
from jax import lax
from jax.experimental import pallas as pl
from jax.experimental.pallas import tpu as pltpu
from jax.experimental.pallas import tpu_sc as plsc

N_NODES = 10000
D = 128
NC = 2
NS = 16
NW = NC * NS
CHUNK = 128
CHUNKS_PER_W = 80
GCHUNKS = 16
E_PAD = NW * CHUNKS_PER_W * CHUNK
N_PAD = 10112
ROWS_PER_TILE = N_PAD // NS
DUMMY_COL = N_NODES

_mesh = plsc.VectorSubcoreMesh(core_axis_name="c", subcore_axis_name="s")


def _zero_acc_slice(zsrc, acc, base):
    full = ROWS_PER_TILE // CHUNK
    for k in range(full):
        pltpu.sync_copy(zsrc, acc.at[pl.ds(base + k * CHUNK, CHUNK)])
    rem = ROWS_PER_TILE - full * CHUNK
    if rem:
        pltpu.sync_copy(zsrc.at[pl.ds(0, rem)],
                        acc.at[pl.ds(base + full * CHUNK, rem)])


@functools.partial(
    pl.kernel,
    mesh=_mesh,
    out_type=jax.ShapeDtypeStruct((NC, N_PAD, D), jnp.float32),
    scratch_types=[
        pltpu.VMEM((CHUNKS_PER_W, CHUNK), jnp.int32),
        pltpu.VMEM((CHUNK, D), jnp.float32),
        pltpu.VMEM((CHUNK, D), jnp.float32),
        pltpu.VMEM_SHARED((N_PAD, D), jnp.float32),
    ],
)
def _deg_kernel(cidx_hbm, out_hbm, cidx_v, obuf, zbuf, acc):
    core = lax.axis_index("c")
    sid = lax.axis_index("s")
    wid = sid * NC + core
    one16 = jnp.ones((16,), jnp.float32)
    z16 = jnp.zeros((16,), jnp.float32)

    def fill(r, carry):
        for k in range(D // 16):
            obuf[r, pl.ds(k * 16, 16)] = one16
            zbuf[r, pl.ds(k * 16, 16)] = z16
        return carry

    lax.fori_loop(0, CHUNK, fill, 0)
    pltpu.sync_copy(cidx_hbm.at[pl.ds(wid * CHUNKS_PER_W, CHUNKS_PER_W)],
                    cidx_v)
    base = sid * ROWS_PER_TILE
    _zero_acc_slice(zbuf, acc, base)
    plsc.subcore_barrier()

    def body(j, carry):
        pltpu.sync_copy(obuf, acc.at[cidx_v.at[j]], add=True)
        return carry

    lax.fori_loop(0, CHUNKS_PER_W, body, 0)
    plsc.subcore_barrier()
    pltpu.sync_copy(acc.at[pl.ds(base, ROWS_PER_TILE)],
                    out_hbm.at[core, pl.ds(base, ROWS_PER_TILE)])


@functools.partial(
    pl.kernel,
    mesh=_mesh,
    out_type=jax.ShapeDtypeStruct((NC, N_PAD, D), jnp.float32),
    scratch_types=[
        pltpu.VMEM((GCHUNKS, CHUNK), jnp.int32),
        pltpu.VMEM((GCHUNKS, CHUNK), jnp.int32),
        pltpu.VMEM((CHUNK, D), jnp.float32),
        pltpu.VMEM((CHUNK, D), jnp.float32),
        pltpu.VMEM_SHARED((N_PAD, D), jnp.float32),
        pltpu.SemaphoreType.DMA,
        pltpu.SemaphoreType.DMA,
    ],
)
def _agg_kernel(hp_hbm, ridx_hbm, cidx_hbm, out_hbm,
                ridx_v, cidx_v, g0, g1, acc, sem0, sem1):
    core = lax.axis_index("c")
    sid = lax.axis_index("s")
    wid = sid * NC + core
    z16 = jnp.zeros((16,), jnp.float32)

    def zrow(r, carry):
        for k in range(D // 16):
            g0[r, pl.ds(k * 16, 16)] = z16
        return carry

    lax.fori_loop(0, CHUNK, zrow, 0)
    base = sid * ROWS_PER_TILE
    _zero_acc_slice(g0, acc, base)
    plsc.subcore_barrier()

    def group(g, carry):
        gbase = wid * CHUNKS_PER_W + g * GCHUNKS
        pltpu.sync_copy(ridx_hbm.at[pl.ds(gbase, GCHUNKS)], ridx_v)
        pltpu.sync_copy(cidx_hbm.at[pl.ds(gbase, GCHUNKS)], cidx_v)
        pltpu.async_copy(hp_hbm.at[ridx_v.at[0]], g0, sem0)

        def body(i, carry2):
            j0 = 2 * i
            pltpu.async_copy(hp_hbm.at[ridx_v.at[j0 + 1]], g1, sem1)
            pltpu.make_async_copy(hp_hbm.at[ridx_v.at[j0]], g0, sem0).wait()
            pltpu.sync_copy(g0, acc.at[cidx_v.at[j0]], add=True)

            @pl.when(i < GCHUNKS // 2 - 1)
            def _prefetch():
                pltpu.async_copy(hp_hbm.at[ridx_v.at[j0 + 2]], g0, sem0)

            pltpu.make_async_copy(hp_hbm.at[ridx_v.at[j0 + 1]], g1, sem1).wait()
            pltpu.sync_copy(g1, acc.at[cidx_v.at[j0 + 1]], add=True)
            return carry2

        lax.fori_loop(0, GCHUNKS // 2, body, 0)
        return carry

    lax.fori_loop(0, CHUNKS_PER_W // GCHUNKS, group, 0)
    plsc.subcore_barrier()
    pltpu.sync_copy(acc.at[pl.ds(base, ROWS_PER_TILE)],
                    out_hbm.at[core, pl.ds(base, ROWS_PER_TILE)])


BLK = 400


def _lin_body(dp_ref, x_ref, w_ref, hp_ref):
    deg = (dp_ref[0].sum(axis=-1) + dp_ref[1].sum(axis=-1)) * (1.0 / D) + 1.0
    dis = lax.rsqrt(deg)
    h = lax.dot_general(x_ref[...], w_ref[...],
                        (((1,), (1,)), ((), ())),
                        preferred_element_type=jnp.float32)
    hp_ref[...] = h * dis[:, None]


_lin = pl.pallas_call(
    _lin_body,
    grid=(N_NODES // BLK,),
    in_specs=[
        pl.BlockSpec((NC, BLK, D), lambda i: (0, i, 0)),
        pl.BlockSpec((BLK, D), lambda i: (i, 0)),
        pl.BlockSpec((D, D), lambda i: (0, 0)),
    ],
    out_specs=pl.BlockSpec((BLK, D), lambda i: (i, 0)),
    out_shape=jax.ShapeDtypeStruct((N_NODES, D), jnp.float32),
)


def _fin_body(dp_ref, a_ref, hp_ref, b_ref, o_ref):
    deg = (dp_ref[0].sum(axis=-1) + dp_ref[1].sum(axis=-1)) * (1.0 / D) + 1.0
    dis = lax.rsqrt(deg)
    s = a_ref[0] + a_ref[1] + hp_ref[...]
    o_ref[...] = s * dis[:, None] + b_ref[...]


_fin = pl.pallas_call(
    _fin_body,
    grid=(N_NODES // BLK,),
    in_specs=[
        pl.BlockSpec((NC, BLK, D), lambda i: (0, i, 0)),
        pl.BlockSpec((NC, BLK, D), lambda i: (0, i, 0)),
        pl.BlockSpec((BLK, D), lambda i: (i, 0)),
        pl.BlockSpec((1, D), lambda i: (0, 0)),
    ],
    out_specs=pl.BlockSpec((BLK, D), lambda i: (i, 0)),
    out_shape=jax.ShapeDtypeStruct((N_NODES, D), jnp.float32),
)


def kernel(x, edge_index, W, b):
    ei = edge_index.astype(jnp.int32)
    row, col = ei[0], ei[1]
    pad = E_PAD - row.shape[0]
    row = jnp.concatenate([row, jnp.zeros((pad,), jnp.int32)])
    col = jnp.concatenate([col, jnp.full((pad,), DUMMY_COL, jnp.int32)])
    ridx = row.reshape(E_PAD // CHUNK, CHUNK)
    cidx = col.reshape(E_PAD // CHUNK, CHUNK)

    deg_parts = _deg_kernel(cidx)
    hp = _lin(deg_parts, x, W)
    acc_parts = _agg_kernel(hp, ridx, cidx)
    return _fin(deg_parts, acc_parts, hp, b.reshape(1, D))


# speedup vs baseline: 15.2532x; 1.4017x over previous
import functools

import jax
import jax.numpy as jnp
from jax import lax
from jax.experimental import pallas as pl
from jax.experimental.pallas import tpu as pltpu
from jax.experimental.pallas import tpu_sc as plsc

N_NODES = 10000
D = 128
NC = 2
NS = 16
NW = NC * NS
CHUNK = 128
CHUNKS_PER_W = 80
GCHUNKS = 8
SUB = 2
CPW_FAST = 136
CPW_SLOW = 24
FAST_CORE = 0
E_PAD = NW * CHUNKS_PER_W * CHUNK
N_PAD = 10240
ROWS_PER_TILE = N_PAD // NS
DUMMY_COL = N_NODES

_mesh = plsc.VectorSubcoreMesh(core_axis_name="c", subcore_axis_name="s")


def _zero_acc_slice(zsrc, acc, base):
    full = ROWS_PER_TILE // CHUNK
    for k in range(full):
        pltpu.sync_copy(zsrc, acc.at[pl.ds(base + k * CHUNK, CHUNK)])
    rem = ROWS_PER_TILE - full * CHUNK
    if rem:
        pltpu.sync_copy(zsrc.at[pl.ds(0, rem)],
                        acc.at[pl.ds(base + full * CHUNK, rem)])


@functools.partial(
    pl.kernel,
    mesh=_mesh,
    out_type=jax.ShapeDtypeStruct((NC, N_PAD, D), jnp.float32),
    scratch_types=[
        pltpu.VMEM((CHUNKS_PER_W, CHUNK), jnp.int32),
        pltpu.VMEM((CHUNK, D), jnp.float32),
        pltpu.VMEM((CHUNK, D), jnp.float32),
        pltpu.VMEM_SHARED((N_PAD, D), jnp.float32),
    ],
)
def _deg_kernel(cidx_hbm, out_hbm, cidx_v, obuf, zbuf, acc):
    core = lax.axis_index("c")
    sid = lax.axis_index("s")
    wid = sid * NC + core
    one16 = jnp.ones((16,), jnp.float32)
    z16 = jnp.zeros((16,), jnp.float32)

    def fill(r, carry):
        for k in range(D // 16):
            obuf[r, pl.ds(k * 16, 16)] = one16
            zbuf[r, pl.ds(k * 16, 16)] = z16
        return carry

    lax.fori_loop(0, CHUNK, fill, 0)
    pltpu.sync_copy(cidx_hbm.at[pl.ds(wid * CHUNKS_PER_W, CHUNKS_PER_W)],
                    cidx_v)
    base = sid * ROWS_PER_TILE
    _zero_acc_slice(zbuf, acc, base)
    plsc.subcore_barrier()

    def body(j, carry):
        pltpu.sync_copy(obuf, acc.at[cidx_v.at[j]], add=True)
        return carry

    lax.fori_loop(0, CHUNKS_PER_W, body, 0)
    plsc.subcore_barrier()
    pltpu.sync_copy(acc.at[pl.ds(base, ROWS_PER_TILE)],
                    out_hbm.at[core, pl.ds(base, ROWS_PER_TILE)])


@functools.partial(
    pl.kernel,
    mesh=_mesh,
    out_type=jax.ShapeDtypeStruct((NC, N_PAD, D), jnp.float32),
    scratch_types=[
        pltpu.VMEM((GCHUNKS, CHUNK), jnp.int32),
        pltpu.VMEM((GCHUNKS, CHUNK), jnp.int32),
        pltpu.VMEM((CHUNK, D), jnp.float32),
        pltpu.VMEM((CHUNK, D), jnp.float32),
        pltpu.VMEM_SHARED((N_PAD, D), jnp.float32),
        pltpu.SemaphoreType.DMA,
        pltpu.SemaphoreType.DMA,
    ],
)
def _agg_kernel(hp_hbm, ridx_hbm, cidx_hbm, out_hbm,
                ridx_v, cidx_v, g0, g1, acc, sem0, sem1):
    core = lax.axis_index("c")
    sid = lax.axis_index("s")
    is_fast = core == FAST_CORE
    my_cpw = jnp.where(is_fast, CPW_FAST, CPW_SLOW)
    chunk0 = jnp.where(is_fast, sid * CPW_FAST,
                       NS * CPW_FAST + sid * CPW_SLOW)
    z16 = jnp.zeros((16,), jnp.float32)

    def zrow(r, carry):
        for k in range(D // 16):
            g0[r, pl.ds(k * 16, 16)] = z16
        return carry

    base = sid * ROWS_PER_TILE
    lax.fori_loop(0, CHUNK, zrow, 0)
    _zero_acc_slice(g0, acc, base)
    plsc.subcore_barrier()

    def start_gather(j, g, sem):
        step = CHUNK // SUB
        for q in range(SUB):
            pltpu.async_copy(hp_hbm.at[ridx_v.at[j, pl.ds(q * step, step)]],
                             g.at[pl.ds(q * step, step)], sem)

    def wait_gather(j, g, sem):
        pltpu.make_async_copy(hp_hbm.at[ridx_v.at[j]], g, sem).wait()

    def group(g, carry):
        gbase = chunk0 + g * GCHUNKS
        pltpu.sync_copy(ridx_hbm.at[pl.ds(gbase, GCHUNKS)], ridx_v)
        pltpu.sync_copy(cidx_hbm.at[pl.ds(gbase, GCHUNKS)], cidx_v)
        start_gather(0, g0, sem0)

        def body(i, carry2):
            j0 = 2 * i
            start_gather(j0 + 1, g1, sem1)
            wait_gather(j0, g0, sem0)
            pltpu.sync_copy(g0, acc.at[cidx_v.at[j0]], add=True)

            @pl.when(i < GCHUNKS // 2 - 1)
            def _prefetch():
                start_gather(j0 + 2, g0, sem0)

            wait_gather(j0 + 1, g1, sem1)
            pltpu.sync_copy(g1, acc.at[cidx_v.at[j0 + 1]], add=True)
            return carry2

        lax.fori_loop(0, GCHUNKS // 2, body, 0)
        return carry

    lax.fori_loop(0, my_cpw // GCHUNKS, group, 0)
    plsc.subcore_barrier()
    pltpu.sync_copy(acc.at[pl.ds(base, ROWS_PER_TILE)],
                    out_hbm.at[core, pl.ds(base, ROWS_PER_TILE)])


BLK = 400


def _mm_body(x_ref, w_ref, h_ref):
    h_ref[...] = lax.dot_general(x_ref[...], w_ref[...],
                                 (((1,), (1,)), ((), ())),
                                 preferred_element_type=jnp.float32)


_mm = pl.pallas_call(
    _mm_body,
    grid=(N_NODES // BLK,),
    in_specs=[
        pl.BlockSpec((BLK, D), lambda i: (i, 0)),
        pl.BlockSpec((D, D), lambda i: (0, 0)),
    ],
    out_specs=pl.BlockSpec((BLK, D), lambda i: (i, 0)),
    out_shape=jax.ShapeDtypeStruct((N_NODES, D), jnp.float32),
)


def _scale_body(dp_ref, h_ref, hp_ref):
    deg = (dp_ref[0].sum(axis=-1) + dp_ref[1].sum(axis=-1)) * (1.0 / D) + 1.0
    dis = lax.rsqrt(deg)
    hp_ref[...] = h_ref[...] * dis[:, None]


_scale = pl.pallas_call(
    _scale_body,
    grid=(N_NODES // BLK,),
    in_specs=[
        pl.BlockSpec((NC, BLK, D), lambda i: (0, i, 0)),
        pl.BlockSpec((BLK, D), lambda i: (i, 0)),
    ],
    out_specs=pl.BlockSpec((BLK, D), lambda i: (i, 0)),
    out_shape=jax.ShapeDtypeStruct((N_NODES, D), jnp.float32),
)


def _fin_body(dp_ref, a_ref, hp_ref, b_ref, o_ref):
    deg = (dp_ref[0].sum(axis=-1) + dp_ref[1].sum(axis=-1)) * (1.0 / D) + 1.0
    dis = lax.rsqrt(deg)
    s = a_ref[0] + a_ref[1] + hp_ref[...]
    o_ref[...] = s * dis[:, None] + b_ref[...]


_fin = pl.pallas_call(
    _fin_body,
    grid=(N_NODES // BLK,),
    in_specs=[
        pl.BlockSpec((NC, BLK, D), lambda i: (0, i, 0)),
        pl.BlockSpec((NC, BLK, D), lambda i: (0, i, 0)),
        pl.BlockSpec((BLK, D), lambda i: (i, 0)),
        pl.BlockSpec((1, D), lambda i: (0, 0)),
    ],
    out_specs=pl.BlockSpec((BLK, D), lambda i: (i, 0)),
    out_shape=jax.ShapeDtypeStruct((N_NODES, D), jnp.float32),
)


def kernel(x, edge_index, W, b):
    ei = edge_index.astype(jnp.int32)
    row, col = ei[0], ei[1]
    pad = E_PAD - row.shape[0]
    row = jnp.concatenate([row, jnp.zeros((pad,), jnp.int32)])
    col = jnp.concatenate([col, jnp.full((pad,), DUMMY_COL, jnp.int32)])
    ridx = row.reshape(E_PAD // CHUNK, CHUNK)
    cidx = col.reshape(E_PAD // CHUNK, CHUNK)

    h = _mm(x, W)
    deg_parts = _deg_kernel(cidx)
    hp = _scale(deg_parts, h)
    acc = _agg_kernel(hp, ridx, cidx)
    return _fin(deg_parts, acc, hp, b.reshape(1, D))
